# baseline (speedup 1.0000x reference)
import functools

import jax
import jax.numpy as jnp
from jax import lax
from jax.experimental import pallas as pl
from jax.experimental.pallas import tpu as pltpu
from jax.experimental.pallas import tpu_sc as plsc

B, D, T = 16384, 128, 1000
NW = 32
ROWS_PER_W = B // NW
CH = 128
NCHUNK = ROWS_PER_W // CH
LANES = 16


def _body(x0_hbm, t_hbm, noise_hbm, ta_hbm, tb_hbm, out_hbm,
          ta_v, tb_v, t_v, ca_v, cb_v, x0_v, nz_v, out_v,
          sem_tab, sem_in, sem_out):
    wid = lax.axis_index("s") * 2 + lax.axis_index("c")
    slab = wid * ROWS_PER_W

    htab_a = pltpu.async_copy(ta_hbm, ta_v, sem_tab)
    htab_b = pltpu.async_copy(tb_hbm, tb_v, sem_tab)
    ht = pltpu.async_copy(t_hbm.at[pl.ds(slab, ROWS_PER_W)], t_v, sem_tab)

    def in_copies(c, slot):
        base = slab + c * CH
        return (
            pltpu.make_async_copy(x0_hbm.at[pl.ds(base, CH)], x0_v.at[slot],
                                  sem_in.at[slot]),
            pltpu.make_async_copy(noise_hbm.at[pl.ds(base, CH)],
                                  nz_v.at[slot], sem_in.at[slot]),
        )

    def start_in(c, slot):
        for cp in in_copies(c, slot):
            cp.start()

    def out_copy(c, slot):
        base = slab + c * CH
        return pltpu.make_async_copy(out_v.at[slot],
                                     out_hbm.at[pl.ds(base, CH)],
                                     sem_out.at[slot])

    start_in(0, 0)
    htab_a.wait()
    htab_b.wait()
    ht.wait()

    def gather_grp(g, _):
        sl = pl.ds(g * LANES, LANES)
        idx = t_v[sl]
        ca_v[sl] = plsc.load_gather(ta_v, [idx])
        cb_v[sl] = plsc.load_gather(tb_v, [idx])
        return 0

    lax.fori_loop(0, ROWS_PER_W // LANES, gather_grp, 0)

    start_in(1, 1)

    def compute(c, slot):
        x0s, nzs, outs = x0_v.at[slot], nz_v.at[slot], out_v.at[slot]

        def row(r, _):
            rsplat = jnp.broadcast_to(c * CH + r, (LANES,))
            av = plsc.load_gather(ca_v, [rsplat])
            bv = plsc.load_gather(cb_v, [rsplat])
            xr, nr, outr = x0s.at[r], nzs.at[r], outs.at[r]
            for j in range(D // LANES):
                sl = pl.ds(j * LANES, LANES)
                outr[sl] = av * xr[sl] + bv * nr[sl]
            return 0

        lax.fori_loop(0, CH, row, 0)

    def chunk(c, _):
        slot = lax.rem(c, 2)
        for cp in in_copies(c, slot):
            cp.wait()

        @pl.when(c >= 2)
        def _():
            out_copy(c - 2, slot).wait()

        compute(c, slot)
        out_copy(c, slot).start()

        @pl.when(c + 2 < NCHUNK)
        def _():
            start_in(c + 2, slot)
        return 0

    lax.fori_loop(0, NCHUNK, chunk, 0)
    out_copy(NCHUNK - 2, 0).wait()
    out_copy(NCHUNK - 1, 1).wait()


def kernel(x0, t, noise, sqrt_alphas_cumprod, sqrt_one_minus_alphas_cumprod):
    mesh = plsc.VectorSubcoreMesh(core_axis_name="c", subcore_axis_name="s")
    f = functools.partial(
        pl.kernel,
        mesh=mesh,
        out_type=jax.ShapeDtypeStruct((B, D), jnp.float32),
        compiler_params=pltpu.CompilerParams(
            needs_layout_passes=False,
            disable_bounds_checks=True,
        ),
        scratch_types=[
            pltpu.VMEM((T,), jnp.float32),
            pltpu.VMEM((T,), jnp.float32),
            pltpu.VMEM((ROWS_PER_W,), jnp.int32),
            pltpu.VMEM((ROWS_PER_W,), jnp.float32),
            pltpu.VMEM((ROWS_PER_W,), jnp.float32),
            pltpu.VMEM((2, CH, D), jnp.float32),
            pltpu.VMEM((2, CH, D), jnp.float32),
            pltpu.VMEM((2, CH, D), jnp.float32),
            pltpu.SemaphoreType.DMA,
            pltpu.SemaphoreType.DMA((2,)),
            pltpu.SemaphoreType.DMA((2,)),
        ],
    )(_body)
    return f(x0, t, noise, sqrt_alphas_cumprod, sqrt_one_minus_alphas_cumprod)


# speedup vs baseline: 1.6038x; 1.6038x over previous
import functools

import jax
import jax.numpy as jnp
from jax import lax
from jax.experimental import pallas as pl
from jax.experimental.pallas import tpu as pltpu
from jax.experimental.pallas import tpu_sc as plsc

B, D, T = 16384, 128, 1000
NW = 32
ROWS_PER_W = B // NW
CH = 64
NCHUNK = ROWS_PER_W // CH
LANES = 16


def _body(x0_hbm, t_hbm, noise_hbm, ta_hbm, tb_hbm, out_hbm,
          ta_v, tb_v, t_v, ca_v, cb_v, x0_v, nz_v, out_v,
          sem_tab, sem_in, sem_out):
    wid = lax.axis_index("s") * 2 + lax.axis_index("c")
    slab = wid * ROWS_PER_W

    htab_a = pltpu.async_copy(ta_hbm, ta_v, sem_tab)
    htab_b = pltpu.async_copy(tb_hbm, tb_v, sem_tab)
    ht = pltpu.async_copy(t_hbm.at[pl.ds(slab, ROWS_PER_W)], t_v, sem_tab)

    def in_copies(c, slot):
        base = slab + c * CH
        return (
            pltpu.make_async_copy(x0_hbm.at[pl.ds(base, CH)], x0_v.at[slot],
                                  sem_in.at[slot]),
            pltpu.make_async_copy(noise_hbm.at[pl.ds(base, CH)],
                                  nz_v.at[slot], sem_in.at[slot]),
        )

    def start_in(c, slot):
        for cp in in_copies(c, slot):
            cp.start()

    def out_copy(c, slot):
        base = slab + c * CH
        return pltpu.make_async_copy(out_v.at[slot],
                                     out_hbm.at[pl.ds(base, CH)],
                                     sem_out.at[slot])

    start_in(0, 0)
    htab_a.wait()
    htab_b.wait()
    ht.wait()

    def gather_grp(g, _):
        sl = pl.ds(g * LANES, LANES)
        idx = t_v[sl]
        ca_v[sl] = plsc.load_gather(ta_v, [idx])
        cb_v[sl] = plsc.load_gather(tb_v, [idx])
        return 0

    lax.fori_loop(0, ROWS_PER_W // LANES, gather_grp, 0)

    start_in(1, 1)

    def compute(c, slot):
        x0s, nzs, outs = x0_v.at[slot], nz_v.at[slot], out_v.at[slot]

        def row(r, _):
            rsplat = jnp.broadcast_to(c * CH + r, (LANES,))
            av = plsc.load_gather(ca_v, [rsplat])
            bv = plsc.load_gather(cb_v, [rsplat])
            xr, nr, outr = x0s.at[r], nzs.at[r], outs.at[r]
            for j in range(D // LANES):
                sl = pl.ds(j * LANES, LANES)
                outr[sl] = av * xr[sl] + bv * nr[sl]
            return 0

        lax.fori_loop(0, CH, row, 0)

    def super_chunk(cc, _):
        for half in range(2):
            c = 2 * cc + half
            for cp in in_copies(c, half):
                cp.wait()

            @pl.when(cc >= 1)
            def _():
                out_copy(c - 2, half).wait()

            compute(c, half)
            out_copy(c, half).start()

            @pl.when(cc < NCHUNK // 2 - 1)
            def _():
                start_in(c + 2, half)
        return 0

    lax.fori_loop(0, NCHUNK // 2, super_chunk, 0)
    out_copy(NCHUNK - 2, 0).wait()
    out_copy(NCHUNK - 1, 1).wait()


def kernel(x0, t, noise, sqrt_alphas_cumprod, sqrt_one_minus_alphas_cumprod):
    mesh = plsc.VectorSubcoreMesh(core_axis_name="c", subcore_axis_name="s")
    f = functools.partial(
        pl.kernel,
        mesh=mesh,
        out_type=jax.ShapeDtypeStruct((B, D), jnp.float32),
        compiler_params=pltpu.CompilerParams(
            needs_layout_passes=False,
            disable_bounds_checks=True,
        ),
        scratch_types=[
            pltpu.VMEM((T,), jnp.float32),
            pltpu.VMEM((T,), jnp.float32),
            pltpu.VMEM((ROWS_PER_W,), jnp.int32),
            pltpu.VMEM((ROWS_PER_W,), jnp.float32),
            pltpu.VMEM((ROWS_PER_W,), jnp.float32),
            pltpu.VMEM((2, CH, D), jnp.float32),
            pltpu.VMEM((2, CH, D), jnp.float32),
            pltpu.VMEM((2, CH, D), jnp.float32),
            pltpu.SemaphoreType.DMA,
            pltpu.SemaphoreType.DMA((2,)),
            pltpu.SemaphoreType.DMA((2,)),
        ],
    )(_body)
    return f(x0, t, noise, sqrt_alphas_cumprod, sqrt_one_minus_alphas_cumprod)


# speedup vs baseline: 1.6507x; 1.0292x over previous
import functools

import jax
import jax.numpy as jnp
from jax import lax
from jax.experimental import pallas as pl
from jax.experimental.pallas import tpu as pltpu
from jax.experimental.pallas import tpu_sc as plsc

B, D, T = 16384, 128, 1000
NW = 32
ROWS_PER_W = B // NW
CH = 128
NCHUNK = ROWS_PER_W // CH
LANES = 16


def _body(x0_hbm, t_hbm, noise_hbm, ta_hbm, tb_hbm, out_hbm,
          ta_v, tb_v, t_v, ca_v, cb_v, x0_v, nz_v, out_v,
          sem_tab, sem_in, sem_out):
    wid = lax.axis_index("s") * 2 + lax.axis_index("c")
    slab = wid * ROWS_PER_W

    htab_a = pltpu.async_copy(ta_hbm, ta_v, sem_tab)
    htab_b = pltpu.async_copy(tb_hbm, tb_v, sem_tab)
    ht = pltpu.async_copy(t_hbm.at[pl.ds(slab, ROWS_PER_W)], t_v, sem_tab)

    def in_copies(c, slot):
        base = slab + c * CH
        return (
            pltpu.make_async_copy(x0_hbm.at[pl.ds(base, CH)], x0_v.at[slot],
                                  sem_in.at[slot]),
            pltpu.make_async_copy(noise_hbm.at[pl.ds(base, CH)],
                                  nz_v.at[slot], sem_in.at[slot]),
        )

    def start_in(c, slot):
        for cp in in_copies(c, slot):
            cp.start()

    def out_copy(c, slot):
        base = slab + c * CH
        return pltpu.make_async_copy(out_v.at[slot],
                                     out_hbm.at[pl.ds(base, CH)],
                                     sem_out.at[slot])

    start_in(0, 0)
    htab_a.wait()
    htab_b.wait()
    ht.wait()

    def gather_grp(g, _):
        sl = pl.ds(g * LANES, LANES)
        idx = t_v[sl]
        ca_v[sl] = plsc.load_gather(ta_v, [idx])
        cb_v[sl] = plsc.load_gather(tb_v, [idx])
        return 0

    lax.fori_loop(0, ROWS_PER_W // LANES, gather_grp, 0)

    start_in(1, 1)

    def compute(c, slot):
        x0s, nzs, outs = x0_v.at[slot], nz_v.at[slot], out_v.at[slot]

        def row(r, _):
            rsplat = jnp.broadcast_to(c * CH + r, (LANES,))
            av = plsc.load_gather(ca_v, [rsplat])
            bv = plsc.load_gather(cb_v, [rsplat])
            xr, nr, outr = x0s.at[r], nzs.at[r], outs.at[r]
            for j in range(D // LANES):
                sl = pl.ds(j * LANES, LANES)
                outr[sl] = av * xr[sl] + bv * nr[sl]
            return 0

        lax.fori_loop(0, CH, row, 0)

    def super_chunk(cc, _):
        for half in range(2):
            c = 2 * cc + half
            for cp in in_copies(c, half):
                cp.wait()

            @pl.when(cc >= 1)
            def _():
                out_copy(c - 2, half).wait()

            compute(c, half)
            out_copy(c, half).start()

            @pl.when(cc < NCHUNK // 2 - 1)
            def _():
                start_in(c + 2, half)
        return 0

    lax.fori_loop(0, NCHUNK // 2, super_chunk, 0)
    out_copy(NCHUNK - 2, 0).wait()
    out_copy(NCHUNK - 1, 1).wait()


def kernel(x0, t, noise, sqrt_alphas_cumprod, sqrt_one_minus_alphas_cumprod):
    mesh = plsc.VectorSubcoreMesh(core_axis_name="c", subcore_axis_name="s")
    f = functools.partial(
        pl.kernel,
        mesh=mesh,
        out_type=jax.ShapeDtypeStruct((B, D), jnp.float32),
        compiler_params=pltpu.CompilerParams(
            needs_layout_passes=False,
            disable_bounds_checks=True,
        ),
        scratch_types=[
            pltpu.VMEM((T,), jnp.float32),
            pltpu.VMEM((T,), jnp.float32),
            pltpu.VMEM((ROWS_PER_W,), jnp.int32),
            pltpu.VMEM((ROWS_PER_W,), jnp.float32),
            pltpu.VMEM((ROWS_PER_W,), jnp.float32),
            pltpu.VMEM((2, CH, D), jnp.float32),
            pltpu.VMEM((2, CH, D), jnp.float32),
            pltpu.VMEM((2, CH, D), jnp.float32),
            pltpu.SemaphoreType.DMA,
            pltpu.SemaphoreType.DMA((2,)),
            pltpu.SemaphoreType.DMA((2,)),
        ],
    )(_body)
    return f(x0, t, noise, sqrt_alphas_cumprod, sqrt_one_minus_alphas_cumprod)
